# Initial kernel scaffold; baseline (speedup 1.0000x reference)
import jax
import jax.numpy as jnp
from jax.experimental import pallas as pl


def kernel(cls_out, box_out):
    raise NotImplementedError("write your pallas kernel here")


# speedup vs baseline: 1.0009x; 1.0009x over previous
import functools

import jax
import jax.numpy as jnp
from jax.experimental import pallas as pl

NUM_CLASSES = 90
NUM_GMM = 3
MAX_DET = 5000
B = 8
S = 64 * 64
SB = 512


def _mixture(x):
    m = x[0]
    v = x[1]
    w = x[2]
    wmax = jnp.max(w, axis=1, keepdims=True)
    e = jnp.exp(w - wmax)
    wts = e / jnp.sum(e, axis=1, keepdims=True)
    wmean = jnp.sum(wts * m, axis=1)
    ua = jnp.sum(wts * jax.nn.sigmoid(v), axis=1)
    ue = jnp.sum(wts * (m - wmean[:, None, :]) ** 2, axis=1)
    return wmean, ua, ue


def _stage1_body(cls_ref, box_ref, sc_ref, cua_ref, cue_ref,
                 bm_ref, bua_ref, bue_ref):
    cm, cua, cue = _mixture(cls_ref[0])
    sc_ref[0] = cm
    cua_ref[0] = cua
    cue_ref[0] = cue
    bm, bua, bue = _mixture(box_ref[0])
    bm_ref[0] = bm
    bua_ref[0] = bua
    bue_ref[0] = bue


def _stage1(cls5, box5):
    grid = (B, S // SB)
    in_specs = [
        pl.BlockSpec((1, 3, NUM_CLASSES, 3, SB), lambda b, s: (b, 0, 0, 0, s)),
        pl.BlockSpec((1, 3, 4, 3, SB), lambda b, s: (b, 0, 0, 0, s)),
    ]
    out_specs = [
        pl.BlockSpec((1, NUM_CLASSES, SB), lambda b, s: (b, 0, s)),
        pl.BlockSpec((1, NUM_CLASSES, SB), lambda b, s: (b, 0, s)),
        pl.BlockSpec((1, NUM_CLASSES, SB), lambda b, s: (b, 0, s)),
        pl.BlockSpec((1, 4, SB), lambda b, s: (b, 0, s)),
        pl.BlockSpec((1, 4, SB), lambda b, s: (b, 0, s)),
        pl.BlockSpec((1, 4, SB), lambda b, s: (b, 0, s)),
    ]
    out_shape = [
        jax.ShapeDtypeStruct((B, NUM_CLASSES, S), jnp.float32),
        jax.ShapeDtypeStruct((B, NUM_CLASSES, S), jnp.float32),
        jax.ShapeDtypeStruct((B, NUM_CLASSES, S), jnp.float32),
        jax.ShapeDtypeStruct((B, 4, S), jnp.float32),
        jax.ShapeDtypeStruct((B, 4, S), jnp.float32),
        jax.ShapeDtypeStruct((B, 4, S), jnp.float32),
    ]
    return pl.pallas_call(
        _stage1_body,
        grid=grid,
        in_specs=in_specs,
        out_specs=out_specs,
        out_shape=out_shape,
    )(cls5, box5)


def _scores_like_ref(cls_out):
    x = jnp.transpose(cls_out, (0, 2, 3, 1))
    mean, var, w = jnp.split(x, 3, axis=-1)
    b, h, wd, ck = w.shape
    c = ck // NUM_GMM
    wts = jax.nn.softmax(w.reshape(b, h, wd, c, NUM_GMM), axis=-1)
    wflat = wts.reshape(b, h, wd, ck)
    weighted_mean = (wflat * mean).reshape(b, h, wd, c, NUM_GMM).sum(-1)
    return weighted_mean.reshape(B, -1)


def kernel(cls_out, box_out):
    cls5 = cls_out.reshape(B, 3, NUM_CLASSES, NUM_GMM, S)
    box5 = box_out.reshape(B, 3, 4, NUM_GMM, S)
    scores, cls_ua, cls_ue, box_m, box_ua, box_ue = _stage1(cls5, box5)

    flat = _scores_like_ref(cls_out)
    vals, i = jax.lax.top_k(flat, MAX_DET)
    s_idx = i // NUM_CLASSES
    c_idx = i % NUM_CLASSES
    j = c_idx * S + s_idx
    cls_topk = vals[..., None]
    cua = jnp.take_along_axis(cls_ua.reshape(B, -1), j, axis=1)[..., None]
    cue = jnp.take_along_axis(cls_ue.reshape(B, -1), j, axis=1)[..., None]
    bm = jnp.take_along_axis(box_m, s_idx[:, None, :], axis=2)
    bua = jnp.take_along_axis(box_ua, s_idx[:, None, :], axis=2)
    bue = jnp.take_along_axis(box_ue, s_idx[:, None, :], axis=2)
    box_topk = jnp.transpose(bm, (0, 2, 1))
    box_ua_t = jnp.max(bua, axis=1)[..., None]
    box_ue_t = jnp.max(bue, axis=1)[..., None]
    return (cls_topk, cua, cue, box_topk, box_ua_t, box_ue_t,
            s_idx, c_idx)


# speedup vs baseline: 3.1300x; 3.1271x over previous
import functools

import jax
import jax.numpy as jnp
from jax import lax
from jax.experimental import pallas as pl
from jax.experimental.pallas import tpu as pltpu
from jax.experimental.pallas import tpu_sc as plsc

NUM_CLASSES = 90
NUM_GMM = 3
K = 5000
B = 8
S = 64 * 64
SB = 512
N = S * NUM_CLASSES
QW = N // 4
WIN = 9216
NWIN = QW // WIN
HB = 2048
CAPW = 4096
CAPF = 4 * CAPW
GL = 1256
GSTRIDE = 1248


def _mixture(x):
    m = x[0]
    v = x[1]
    w = x[2]
    wmax = jnp.max(w, axis=1, keepdims=True)
    e = jnp.exp(w - wmax)
    wts = e / jnp.sum(e, axis=1, keepdims=True)
    wmean = jnp.sum(wts * m, axis=1)
    ua = jnp.sum(wts * jax.nn.sigmoid(v), axis=1)
    ue = jnp.sum(wts * (m - wmean[:, None, :]) ** 2, axis=1)
    return wmean, ua, ue


def _stage1_body(cls_ref, box_ref, cua_ref, cue_ref, bm_ref, bua_ref, bue_ref):
    _, cua, cue = _mixture(cls_ref[0])
    cua_ref[0] = cua
    cue_ref[0] = cue
    bm, bua, bue = _mixture(box_ref[0])
    bm_ref[0] = bm
    bua_ref[0] = bua
    bue_ref[0] = bue


def _stage1(cls5, box5):
    grid = (B, S // SB)
    cspec = pl.BlockSpec((1, NUM_CLASSES, SB), lambda b, s: (b, 0, s))
    bspec = pl.BlockSpec((1, 4, SB), lambda b, s: (b, 0, s))
    return pl.pallas_call(
        _stage1_body,
        grid=grid,
        in_specs=[
            pl.BlockSpec((1, 3, NUM_CLASSES, 3, SB), lambda b, s: (b, 0, 0, 0, s)),
            pl.BlockSpec((1, 3, 4, 3, SB), lambda b, s: (b, 0, 0, 0, s)),
        ],
        out_specs=[cspec, cspec, bspec, bspec, bspec],
        out_shape=[
            jax.ShapeDtypeStruct((B, NUM_CLASSES, S), jnp.float32),
            jax.ShapeDtypeStruct((B, NUM_CLASSES, S), jnp.float32),
            jax.ShapeDtypeStruct((B, 4, S), jnp.float32),
            jax.ShapeDtypeStruct((B, 4, S), jnp.float32),
            jax.ShapeDtypeStruct((B, 4, S), jnp.float32),
        ],
    )(cls5, box5)


def _scores_like_ref(cls_out):
    x = jnp.transpose(cls_out, (0, 2, 3, 1))
    mean, var, w = jnp.split(x, 3, axis=-1)
    b, h, wd, ck = w.shape
    c = ck // NUM_GMM
    wts = jax.nn.softmax(w.reshape(b, h, wd, c, NUM_GMM), axis=-1)
    wflat = wts.reshape(b, h, wd, ck)
    weighted_mean = (wflat * mean).reshape(b, h, wd, c, NUM_GMM).sum(-1)
    return weighted_mean.reshape(B * N)


_mesh = plsc.VectorSubcoreMesh(core_axis_name="c", subcore_axis_name="s")


def _sc_body(sco, cua, cue, bm, bua, bue,
             o_val, o_cua, o_cue, o_box, o_bua, o_bue, o_idx, o_cls,
             v_win, v_hp, v_hs, v_cu, v_ci, v_fu, v_fi, v_rh, v_ro, v_cn,
             g_i, g_w1, g_w2, f_a, f_b, f_c, f_d, f_o,
             sp_hist, sp_cu, sp_ci, sp_cnt, sp_su, sp_si, sem):
    cid = lax.axis_index("c")
    sid = lax.axis_index("s")
    lb = sid // 4
    q = sid % 4
    b = cid * 4 + lb
    lanes = lax.iota(jnp.int32, 16)
    lm1 = jnp.maximum(lanes - 1, 0)
    lp1 = jnp.minimum(lanes + 1, 15)
    zeros16 = jnp.zeros((16,), jnp.int32)
    ones16 = jnp.ones((16,), jnp.int32)
    base = b * N + q * QW

    def monokey(f32v):
        bb = plsc.bitcast(f32v, jnp.int32)
        sgn = lax.shift_right_arithmetic(bb, 31)
        kk = bb ^ lax.shift_right_logical(sgn, 1)
        return kk ^ jnp.int32(-2147483648)

    def zhp(t, _):
        v_hp[pl.ds(pl.multiple_of(16 * t, 8), 16)] = zeros16
        return 0
    lax.fori_loop(0, 2048, zhp, 0)

    lane_off = lanes * HB
    for w in range(NWIN):
        pltpu.sync_copy(sco.at[pl.ds(pl.multiple_of(base + w * WIN, 8), WIN)], v_win)

        def hbody(t, _):
            ub = monokey(v_win[pl.ds(pl.multiple_of(16 * t, 8), 16)])
            bkt = lax.shift_right_logical(ub, 21)
            plsc.addupdate_scatter(v_hp, [lane_off + bkt], ones16)
            return 0
        lax.fori_loop(0, WIN // 16, hbody, 0)

    def hred(g, _):
        acc = v_hp[pl.ds(pl.multiple_of(16 * g, 8), 16)]
        for l in range(1, 16):
            acc = acc + v_hp[pl.ds(pl.multiple_of(l * HB + 16 * g, 8), 16)]
        v_hs[pl.ds(pl.multiple_of(16 * g, 8), 16)] = acc
        return 0
    lax.fori_loop(0, HB // 16, hred, 0)
    pltpu.sync_copy(v_hs, sp_hist.at[pl.ds(pl.multiple_of(sid * HB, 8), HB)])
    plsc.subcore_barrier()

    for qq in range(4):
        pltpu.sync_copy(sp_hist.at[pl.ds(pl.multiple_of((lb * 4 + qq) * HB, 8), HB)], v_hp.at[pl.ds(qq * HB, HB)])

    def bsum(g, _):
        acc = v_hp[pl.ds(pl.multiple_of(16 * g, 8), 16)]
        for qq in range(1, 4):
            acc = acc + v_hp[pl.ds(pl.multiple_of(qq * HB + 16 * g, 8), 16)]
        v_hs[pl.ds(pl.multiple_of(16 * g, 8), 16)] = acc
        return 0
    lax.fori_loop(0, HB // 16, bsum, 0)

    def tscan(g, carry):
        acc, bstar, found = carry
        gg = (HB // 16 - 1) - g
        v = v_hs[pl.ds(pl.multiple_of(16 * gg, 8), 16)]
        rv = lax.rev(v, (0,))
        cs = plsc.cumsum(rv)
        csf = cs + acc
        meets = csf >= K
        anyv = jnp.any(meets)
        j = plsc.all_reduce_ffs(meets)
        bucket = 16 * gg + 15 - j
        newb = jnp.where(found | (~anyv), bstar, bucket)
        tot = lax.reduce_max(cs, (0,))
        return acc + tot, newb, found | anyv

    _, bstar_v, _ = lax.fori_loop(
        0, HB // 16, tscan,
        (jnp.zeros((16,), jnp.int32), jnp.zeros((16,), jnp.int32),
         jnp.zeros((16,), jnp.bool_)))

    wv = jnp.zeros((16,), jnp.int32)
    for w in range(NWIN):
        pltpu.sync_copy(sco.at[pl.ds(pl.multiple_of(base + w * WIN, 8), WIN)], v_win)

        def ebody(t, wv):
            ub = monokey(v_win[pl.ds(pl.multiple_of(16 * t, 8), 16)])
            bkt = lax.shift_right_logical(ub, 21)
            msk = bkt >= bstar_v
            cs = plsc.cumsum(msk.astype(jnp.int32))
            pos = wv + cs - 1
            msk2 = msk & (pos < CAPW)
            posc = jnp.where(msk2, pos, 0)
            ck = ub ^ jnp.int32(-1)
            plsc.store_scatter(v_cu, [posc], ck, mask=msk2)
            iv = (q * QW + w * WIN) + 16 * t + lanes
            plsc.store_scatter(v_ci, [posc], iv, mask=msk2)
            return wv + lax.reduce_max(cs, (0,))
        wv = lax.fori_loop(0, WIN // 16, ebody, wv)

    cw = jnp.minimum(lax.reduce_max(wv, (0,)), CAPW)
    pad = (8 - lax.rem(cw, 8)) & 7
    pmask = (lanes < pad) & ((cw + lanes) < CAPW)
    ppos = jnp.where(pmask, cw + lanes, 0)
    plsc.store_scatter(v_cu, [ppos], jnp.full((16,), -1, jnp.int32), mask=pmask)
    plsc.store_scatter(v_ci, [ppos], zeros16, mask=pmask)
    cwp = cw + pad
    v_cn[pl.ds(0, 16)] = jnp.full((16,), cwp, jnp.int32)
    pltpu.sync_copy(v_cn.at[pl.ds(0, 16)], sp_cnt.at[pl.ds(pl.multiple_of(sid * 16, 8), 16)])
    pltpu.sync_copy(v_cu, sp_cu.at[pl.ds(pl.multiple_of(sid * CAPW, 8), CAPW)])
    pltpu.sync_copy(v_ci, sp_ci.at[pl.ds(pl.multiple_of(sid * CAPW, 8), CAPW)])
    plsc.subcore_barrier()

    @pl.when(q == 0)
    def _sort():
        for qq in range(4):
            pltpu.sync_copy(sp_cnt.at[pl.ds(pl.multiple_of((lb * 4 + qq) * 16, 8), 16)], v_cn.at[pl.ds(16 * qq, 16)])
        c0 = lax.reduce_max(v_cn[pl.ds(0, 16)], (0,))
        c1 = lax.reduce_max(v_cn[pl.ds(16, 16)], (0,))
        c2 = lax.reduce_max(v_cn[pl.ds(32, 16)], (0,))
        c3 = lax.reduce_max(v_cn[pl.ds(48, 16)], (0,))
        offs = [jnp.int32(0), c0, c0 + c1, c0 + c1 + c2]
        ct = c0 + c1 + c2 + c3
        for qq in range(4):
            pltpu.sync_copy(sp_cu.at[pl.ds(pl.multiple_of((lb * 4 + qq) * CAPW, 8), CAPW)], v_fu.at[pl.ds(pl.multiple_of(offs[qq], 8), CAPW)])
            pltpu.sync_copy(sp_ci.at[pl.ds(pl.multiple_of((lb * 4 + qq) * CAPW, 8), CAPW)], v_fi.at[pl.ds(pl.multiple_of(offs[qq], 8), CAPW)])
        nv = (ct + 15) // 16
        tmask = (lanes < (16 * nv - ct)) & ((ct + lanes) < CAPF)
        tpos = jnp.where(tmask, ct + lanes, 0)
        plsc.store_scatter(v_fu, [tpos], jnp.full((16,), -1, jnp.int32),
                           mask=tmask)
        plsc.store_scatter(v_fi, [tpos], zeros16, mask=tmask)

        lane_off256 = lanes * 256
        for p in range(4):
            if p % 2 == 0:
                su, si_, du, di = 0, 0, 0, 16384
                sref_u, sref_i, dref_u, dref_i = v_fu, v_fi, v_hp, v_hp
            else:
                su, si_, du, di = 0, 16384, 0, 0
                sref_u, sref_i, dref_u, dref_i = v_hp, v_hp, v_fu, v_fi

            def zrh(t, _):
                v_rh[pl.ds(pl.multiple_of(16 * t, 8), 16)] = zeros16
                return 0
            lax.fori_loop(0, 256, zrh, 0)

            def rhist(t, _):
                ck = sref_u[pl.ds(pl.multiple_of(su + 16 * t, 8), 16)]
                d = lax.shift_right_logical(ck, 8 * p) & 255
                plsc.addupdate_scatter(v_rh, [lane_off256 + d], ones16)
                return 0
            lax.fori_loop(0, nv, rhist, 0)

            for g in range(16):
                acc = v_rh[pl.ds(pl.multiple_of(16 * g, 8), 16)]
                for l in range(1, 16):
                    acc = acc + v_rh[pl.ds(pl.multiple_of(l * 256 + 16 * g, 8), 16)]
                v_ro[pl.ds(pl.multiple_of(16 * g, 8), 16)] = acc
            carry = jnp.int32(0)
            for g in range(16):
                v = v_ro[pl.ds(pl.multiple_of(16 * g, 8), 16)]
                cs = plsc.cumsum(v)
                v_ro[pl.ds(pl.multiple_of(16 * g, 8), 16)] = cs - v + carry
                carry = carry + lax.reduce_max(cs, (0,))

            def perm(t, _):
                ck = sref_u[pl.ds(pl.multiple_of(su + 16 * t, 8), 16)]
                ci = sref_i[pl.ds(pl.multiple_of(si_ + 16 * t, 8), 16)]
                d = lax.shift_right_logical(ck, 8 * p) & 255
                key = d * 32 + lanes
                sk, sl = plsc.sort_key_val(key, lanes)
                sd = lax.shift_right_logical(sk, 5)
                v_cn[pl.ds(0, 16)] = sd
                prev = plsc.load_gather(v_cn.at[pl.ds(0, 16)], [lm1])
                isstart = (lanes == 0) | (sd != prev)
                runstart = plsc.cummax(jnp.where(isstart, lanes, 0))
                segrank = lanes - runstart
                plsc.store_scatter(v_cn.at[pl.ds(16, 16)], [sl], segrank)
                r = v_cn[pl.ds(16, 16)]
                nxt = plsc.load_gather(v_cn.at[pl.ds(0, 16)], [lp1])
                isend = (lanes == 15) | (sd != nxt)
                plsc.store_scatter(v_cn.at[pl.ds(32, 16)], [sl],
                                   isend.astype(jnp.int32))
                lastm = v_cn[pl.ds(32, 16)] != 0
                bs = plsc.load_gather(v_ro, [d])
                pos = bs + r
                plsc.store_scatter(dref_u, [du + pos], ck)
                plsc.store_scatter(dref_i, [di + pos], ci)
                plsc.addupdate_scatter(v_ro, [d], r + 1, mask=lastm)
                return 0
            lax.fori_loop(0, nv, perm, 0)

        pltpu.sync_copy(v_fu.at[pl.ds(0, K)], sp_su.at[pl.ds(pl.multiple_of(lb * K, 8), K)])
        pltpu.sync_copy(v_fi.at[pl.ds(0, K)], sp_si.at[pl.ds(pl.multiple_of(lb * K, 8), K)])
    plsc.subcore_barrier()

    start = GSTRIDE * q
    g_i[pl.ds(GSTRIDE, 16)] = zeros16
    g_i[pl.ds(GSTRIDE + 16, 16)] = zeros16
    pltpu.sync_copy(sp_si.at[pl.ds(pl.multiple_of(lb * K + start, 8), GL)], g_i.at[pl.ds(0, GL)])
    pltpu.sync_copy(sp_su.at[pl.ds(pl.multiple_of(lb * K + start, 8), GL)], g_w1.at[pl.ds(0, GL)])
    NV = 80

    def vals(t, _):
        ck = g_w1[pl.ds(pl.multiple_of(16 * t, 8), 16)]
        ub = ck ^ jnp.int32(-1)
        ui = ub ^ jnp.int32(-2147483648)
        sgn = lax.shift_right_arithmetic(ui, 31)
        bb = ui ^ lax.shift_right_logical(sgn, 1)
        f_a[pl.ds(pl.multiple_of(16 * t, 8), 16)] = plsc.bitcast(bb, jnp.float32)
        return 0
    lax.fori_loop(0, NV, vals, 0)
    obase = b * K + start
    pltpu.sync_copy(f_a.at[pl.ds(0, GL)], o_val.at[pl.ds(pl.multiple_of(obase, 8), GL)])

    def scsplit(t, _):
        iv = g_i[pl.ds(pl.multiple_of(16 * t, 8), 16)]
        s = iv // NUM_CLASSES
        g_w1[pl.ds(pl.multiple_of(16 * t, 8), 16)] = s
        g_w2[pl.ds(pl.multiple_of(16 * t, 8), 16)] = iv - s * NUM_CLASSES
        return 0
    lax.fori_loop(0, NV, scsplit, 0)
    pltpu.sync_copy(g_w1.at[pl.ds(0, GL)], o_idx.at[pl.ds(pl.multiple_of(obase, 8), GL)])
    pltpu.sync_copy(g_w2.at[pl.ds(0, GL)], o_cls.at[pl.ds(pl.multiple_of(obase, 8), GL)])

    def gather_from(tab, idx_ref, dst):
        hs = []
        for t in range(NV // 8):
            hs.append(pltpu.async_copy(
                tab.at[idx_ref.at[pl.ds(128 * t, 128)]],
                dst.at[pl.ds(128 * t, 128)], sem))
        for h in hs:
            h.wait()

    def clsidx(t, _):
        g_i[pl.ds(pl.multiple_of(16 * t, 8), 16)] = (b * N
                                  + g_w2[pl.ds(pl.multiple_of(16 * t, 8), 16)] * S
                                  + g_w1[pl.ds(pl.multiple_of(16 * t, 8), 16)])
        return 0
    lax.fori_loop(0, NV, clsidx, 0)
    gather_from(cua, g_i, f_a)
    pltpu.sync_copy(f_a.at[pl.ds(0, GL)], o_cua.at[pl.ds(pl.multiple_of(obase, 8), GL)])
    gather_from(cue, g_i, f_a)
    pltpu.sync_copy(f_a.at[pl.ds(0, GL)], o_cue.at[pl.ds(pl.multiple_of(obase, 8), GL)])

    fbufs = [f_a, f_b, f_c, f_d]

    def boxidx(d):
        def body(t, _):
            g_i[pl.ds(pl.multiple_of(16 * t, 8), 16)] = ((b * 4 + d) * S
                                      + g_w1[pl.ds(pl.multiple_of(16 * t, 8), 16)])
            return 0
        lax.fori_loop(0, NV, body, 0)

    for d in range(4):
        boxidx(d)
        gather_from(bm, g_i, fbufs[d])

    def interleave(t, _):
        posb = 4 * (16 * t + lanes)
        for d in range(4):
            plsc.store_scatter(f_o, [posb + d], fbufs[d][pl.ds(pl.multiple_of(16 * t, 8), 16)])
        return 0
    lax.fori_loop(0, NV, interleave, 0)
    pltpu.sync_copy(f_o.at[pl.ds(0, 4 * GL)],
                    o_box.at[pl.ds(pl.multiple_of(b * 4 * K + 4 * start, 8), 4 * GL)])

    for tab, out in ((bua, o_bua), (bue, o_bue)):
        for d in range(4):
            boxidx(d)
            gather_from(tab, g_i, fbufs[d])

        def bmax(t, _):
            mx = jnp.maximum(
                jnp.maximum(f_a[pl.ds(pl.multiple_of(16 * t, 8), 16)], f_b[pl.ds(pl.multiple_of(16 * t, 8), 16)]),
                jnp.maximum(f_c[pl.ds(pl.multiple_of(16 * t, 8), 16)], f_d[pl.ds(pl.multiple_of(16 * t, 8), 16)]))
            f_o[pl.ds(pl.multiple_of(16 * t, 8), 16)] = mx
            return 0
        lax.fori_loop(0, NV, bmax, 0)
        pltpu.sync_copy(f_o.at[pl.ds(0, GL)], out.at[pl.ds(pl.multiple_of(obase, 8), GL)])


@functools.partial(
    pl.kernel,
    mesh=_mesh,
    compiler_params=pltpu.CompilerParams(needs_layout_passes=False),
    out_type=[
        jax.ShapeDtypeStruct((B * K,), jnp.float32),
        jax.ShapeDtypeStruct((B * K,), jnp.float32),
        jax.ShapeDtypeStruct((B * K,), jnp.float32),
        jax.ShapeDtypeStruct((B * 4 * K,), jnp.float32),
        jax.ShapeDtypeStruct((B * K,), jnp.float32),
        jax.ShapeDtypeStruct((B * K,), jnp.float32),
        jax.ShapeDtypeStruct((B * K,), jnp.int32),
        jax.ShapeDtypeStruct((B * K,), jnp.int32),
    ],
    scratch_types=[
        pltpu.VMEM((WIN,), jnp.float32),
        pltpu.VMEM((16 * HB,), jnp.int32),
        pltpu.VMEM((HB,), jnp.int32),
        pltpu.VMEM((CAPW,), jnp.int32),
        pltpu.VMEM((CAPW,), jnp.int32),
        pltpu.VMEM((CAPF,), jnp.int32),
        pltpu.VMEM((CAPF,), jnp.int32),
        pltpu.VMEM((16 * 256,), jnp.int32),
        pltpu.VMEM((256,), jnp.int32),
        pltpu.VMEM((64,), jnp.int32),
        pltpu.VMEM((1280,), jnp.int32),
        pltpu.VMEM((1280,), jnp.int32),
        pltpu.VMEM((1280,), jnp.int32),
        pltpu.VMEM((1280,), jnp.float32),
        pltpu.VMEM((1280,), jnp.float32),
        pltpu.VMEM((1280,), jnp.float32),
        pltpu.VMEM((1280,), jnp.float32),
        pltpu.VMEM((5120,), jnp.float32),
        pltpu.VMEM_SHARED((16 * HB,), jnp.int32),
        pltpu.VMEM_SHARED((16 * CAPW,), jnp.int32),
        pltpu.VMEM_SHARED((16 * CAPW,), jnp.int32),
        pltpu.VMEM_SHARED((16 * 16,), jnp.int32),
        pltpu.VMEM_SHARED((4 * K,), jnp.int32),
        pltpu.VMEM_SHARED((4 * K,), jnp.int32),
        pltpu.SemaphoreType.DMA,
    ],
)
def _sc_topk(*args):
    _sc_body(*args)


def kernel(cls_out, box_out):
    cls5 = cls_out.reshape(B, 3, NUM_CLASSES, NUM_GMM, S)
    box5 = box_out.reshape(B, 3, 4, NUM_GMM, S)
    cls_ua, cls_ue, box_m, box_ua, box_ue = _stage1(cls5, box5)
    sco = _scores_like_ref(cls_out)
    o_val, o_cua, o_cue, o_box, o_bua, o_bue, o_idx, o_cls = _sc_topk(
        sco, cls_ua.reshape(-1), cls_ue.reshape(-1),
        box_m.reshape(-1), box_ua.reshape(-1), box_ue.reshape(-1))
    return (
        o_val.reshape(B, K, 1),
        o_cua.reshape(B, K, 1),
        o_cue.reshape(B, K, 1),
        o_box.reshape(B, K, 4),
        o_bua.reshape(B, K, 1),
        o_bue.reshape(B, K, 1),
        o_idx.reshape(B, K),
        o_cls.reshape(B, K),
    )
